# baseline (speedup 1.0000x reference)
import jax
import jax.numpy as jnp
from jax import lax
from jax.experimental import pallas as pl
from jax.experimental.pallas import tpu as pltpu
from jax.experimental.pallas import tpu_sc as plsc

B, H, S, D = 16, 8, 2048, 128
ROWS = B * H
NTILE = 32
ROWS_PER_TILE = ROWS // NTILE
CH = 256
NCH = S // CH
NBUF = 4


def _sc_body(k_ref, v_ref, ko_ref, vo_ref, b0, b1, b2, b3, sems):
    c = lax.axis_index("c")
    s = lax.axis_index("s")
    base = (c * 16 + s) * ROWS_PER_TILE
    bufs = (b0, b1, b2, b3)
    n = ROWS_PER_TILE * NCH

    def sl(i):
        row = base + i // NCH
        off = (i % NCH) * CH
        return (row, pl.ds(off, CH), slice(None))

    def in_copy(t, i, b):
        src = (k_ref, v_ref)[t]
        return pltpu.make_async_copy(src.at[sl(i)], bufs[b], sems.at[b])

    def out_copy(t, i, b):
        dst = (ko_ref, vo_ref)[t]
        return pltpu.make_async_copy(bufs[b], dst.at[sl(i)], sems.at[NBUF + b])

    def tc_of(g):
        return g % 2, g // 2

    total = 2 * n
    LOOK = 2
    for g in range(min(LOOK, total)):
        t, i = tc_of(g)
        in_copy(t, i, g % NBUF).start()
    for g in range(total):
        b = g % NBUF
        t, i = tc_of(g)
        in_copy(t, i, b).wait()
        out_copy(t, i, b).start()
        j = g + LOOK
        if j < total:
            bj = j % NBUF
            tj, ij = tc_of(j)
            if j >= NBUF:
                tp, ip = tc_of(j - NBUF)
                out_copy(tp, ip, bj).wait()
            in_copy(tj, ij, bj).start()
    for g in range(total - NBUF, total):
        b = g % NBUF
        t, i = tc_of(g)
        out_copy(t, i, b).wait()


def kernel(k_val, v_val, k_cache, v_cache):
    k2 = k_val.reshape(ROWS, S, D)
    v2 = v_val.reshape(ROWS, S, D)
    fn = pl.kernel(
        _sc_body,
        out_type=[jax.ShapeDtypeStruct((ROWS, S, D), jnp.float32)] * 2,
        mesh=plsc.VectorSubcoreMesh(core_axis_name="c", subcore_axis_name="s"),
        scratch_types=[pltpu.MemorySpace.VMEM((CH, D), jnp.float32)] * NBUF
        + [pltpu.SemaphoreType.DMA((2 * NBUF,))],
    )
    ko, vo = fn(k2, v2)
    return ko.reshape(B, H, S, D), vo.reshape(B, H, S, D)


# speedup vs baseline: 1.1045x; 1.1045x over previous
import jax
import jax.numpy as jnp
from jax import lax
from jax.experimental import pallas as pl
from jax.experimental.pallas import tpu as pltpu
from jax.experimental.pallas import tpu_sc as plsc

B, H, S, D = 16, 8, 2048, 128
ROWS = B * H
NTILE = 32
ROWS_PER_TILE = ROWS // NTILE
CH = 256
NCH = S // CH
NBUF = 4
LOOK = 2
BR = 4


def _sc_body(src, dst, b0, b1, b2, b3, sems):
    c = lax.axis_index("c")
    s = lax.axis_index("s")
    base = (c * 16 + s) * ROWS_PER_TILE
    bufs = (b0, b1, b2, b3)
    n = ROWS_PER_TILE * NCH

    def sl(i):
        row = base + i // NCH
        off = (i % NCH) * CH
        return (row, pl.ds(off, CH), slice(None))

    def in_copy(i, b):
        return pltpu.make_async_copy(src.at[sl(i)], bufs[b], sems.at[b])

    def out_copy(i, b):
        return pltpu.make_async_copy(bufs[b], dst.at[sl(i)], sems.at[NBUF + b])

    for g in range(min(LOOK, n)):
        in_copy(g, g % NBUF).start()
    for g in range(n):
        b = g % NBUF
        in_copy(g, b).wait()
        out_copy(g, b).start()
        j = g + LOOK
        if j < n:
            bj = j % NBUF
            if j >= NBUF:
                out_copy(j - NBUF, bj).wait()
            in_copy(j, bj).start()
    for g in range(n - NBUF, n):
        out_copy(g, g % NBUF).wait()


def _tc_body(v_ref, vo_ref):
    vo_ref[...] = v_ref[...]


def kernel(k_val, v_val, k_cache, v_cache):
    k2 = k_val.reshape(ROWS, S, D)
    v2 = v_val.reshape(ROWS, S, D)
    sc_fn = pl.kernel(
        _sc_body,
        out_type=jax.ShapeDtypeStruct((ROWS, S, D), jnp.float32),
        mesh=plsc.VectorSubcoreMesh(core_axis_name="c", subcore_axis_name="s"),
        scratch_types=[pltpu.MemorySpace.VMEM((CH, D), jnp.float32)] * NBUF
        + [pltpu.SemaphoreType.DMA((2 * NBUF,))],
    )
    ko = sc_fn(k2)
    spec = pl.BlockSpec((BR, S, D), lambda i: (i, 0, 0))
    vo = pl.pallas_call(
        _tc_body,
        grid=(ROWS // BR,),
        in_specs=[spec],
        out_specs=spec,
        out_shape=jax.ShapeDtypeStruct((ROWS, S, D), jnp.float32),
    )(v2)
    return ko.reshape(B, H, S, D), vo.reshape(B, H, S, D)


# speedup vs baseline: 1.2699x; 1.1497x over previous
import jax
import jax.numpy as jnp
from jax.experimental import pallas as pl
from jax.experimental.pallas import tpu as pltpu

B, H, S, D = 16, 8, 2048, 128
ROWS = B * H
NBUF = 16
LOOK = 8


def _body(k_ref, v_ref, ko_ref, vo_ref, *rest):
    bufs = rest[:NBUF]
    sems = rest[NBUF]

    def in_copy(g, b):
        src = (k_ref, v_ref)[g % 2]
        return pltpu.make_async_copy(src.at[g // 2], bufs[b], sems.at[b])

    def out_copy(g, b):
        dst = (ko_ref, vo_ref)[g % 2]
        return pltpu.make_async_copy(bufs[b], dst.at[g // 2], sems.at[NBUF + b])

    total = 2 * ROWS
    for g in range(LOOK):
        in_copy(g, g % NBUF).start()
    for g in range(total):
        b = g % NBUF
        in_copy(g, b).wait()
        out_copy(g, b).start()
        j = g + LOOK
        if j < total:
            bj = j % NBUF
            if j >= NBUF:
                out_copy(j - NBUF, bj).wait()
            in_copy(j, bj).start()
    for g in range(total - NBUF, total):
        out_copy(g, g % NBUF).wait()


def kernel(k_val, v_val, k_cache, v_cache):
    k2 = k_val.reshape(ROWS, S, D)
    v2 = v_val.reshape(ROWS, S, D)
    out = pl.pallas_call(
        _body,
        in_specs=[pl.BlockSpec(memory_space=pl.ANY)] * 2,
        out_specs=[pl.BlockSpec(memory_space=pl.ANY)] * 2,
        out_shape=[jax.ShapeDtypeStruct((ROWS, S, D), jnp.float32)] * 2,
        scratch_shapes=[pltpu.VMEM((S, D), jnp.float32)] * NBUF
        + [pltpu.SemaphoreType.DMA((2 * NBUF,))],
    )(k2, v2)
    return out[0].reshape(B, H, S, D), out[1].reshape(B, H, S, D)
